# Initial kernel scaffold; baseline (speedup 1.0000x reference)
import jax
import jax.numpy as jnp
from jax.experimental import pallas as pl


def kernel(x, edge_index, W1, b1, W2, b2, Wfc, bfc):
    raise NotImplementedError("write your pallas kernel here")


# speedup vs baseline: 19.6129x; 19.6129x over previous
import functools

import jax
import jax.numpy as jnp
from jax import lax
from jax.experimental import pallas as pl
from jax.experimental.pallas import tpu as pltpu
from jax.experimental.pallas import tpu_sc as plsc

N = 10000
D = 128
C = 16
E = 320000
NC, NS = 2, 16
NW = NC * NS
EPT = E // NW
K = 80
CH = EPT // K
NP = 10240
RPT = NP // NS
ZR = 128
NDP = 10240
DPT = NDP // NS

BR = 1000

_mesh = plsc.VectorSubcoreMesh(core_axis_name="c", subcore_axis_name="s")


@functools.partial(
    pl.kernel,
    mesh=_mesh,
    out_type=jax.ShapeDtypeStruct((NC, NDP), jnp.float32),
    scratch_types=[
        pltpu.VMEM_SHARED((NDP,), jnp.float32),
        pltpu.VMEM((CH, K), jnp.int32),
        pltpu.VMEM((DPT,), jnp.float32),
        pltpu.VMEM((K,), jnp.float32),
    ],
)
def _sc_degree(dst_hbm, out_hbm, acc, dst_l, zbuf, ones_l):
    cid = lax.axis_index("c")
    sid = lax.axis_index("s")
    wid = cid * NS + sid
    for i in range(DPT // 16):
        zbuf[pl.ds(i * 16, 16)] = jnp.zeros((16,), jnp.float32)
    for i in range(K // 16):
        ones_l[pl.ds(i * 16, 16)] = jnp.ones((16,), jnp.float32)
    pltpu.sync_copy(zbuf, acc.at[pl.ds(sid * DPT, DPT)])
    pltpu.sync_copy(dst_hbm.at[wid], dst_l)
    plsc.subcore_barrier()

    def step(j, carry):
        pltpu.sync_copy(ones_l, acc.at[dst_l.at[j]], add=True)
        return carry

    lax.fori_loop(0, CH, step, 0)
    plsc.subcore_barrier()
    pltpu.sync_copy(acc.at[pl.ds(sid * DPT, DPT)],
                    out_hbm.at[cid, pl.ds(sid * DPT, DPT)])


@functools.partial(
    pl.kernel,
    mesh=_mesh,
    out_type=jax.ShapeDtypeStruct((NC, NP, D), jnp.float32),
    scratch_types=[
        pltpu.VMEM_SHARED((NP, D), jnp.float32),
        pltpu.VMEM((CH, K), jnp.int32),
        pltpu.VMEM((CH, K), jnp.int32),
        pltpu.VMEM((K, D), jnp.float32),
        pltpu.SemaphoreType.DMA,
    ],
)
def _sc_gather_scatter(table_hbm, src_hbm, dst_hbm, out_hbm,
                       acc, src_l, dst_l, rows, sem):
    cid = lax.axis_index("c")
    sid = lax.axis_index("s")
    wid = cid * NS + sid

    def zfill(i, carry):
        for c in range(D // 16):
            rows[i, pl.ds(c * 16, 16)] = jnp.zeros((16,), jnp.float32)
        return carry

    lax.fori_loop(0, K, zfill, 0)
    for k in range(RPT // K):
        pltpu.sync_copy(rows, acc.at[pl.ds(sid * RPT + k * K, K)])
    pltpu.sync_copy(src_hbm.at[wid], src_l)
    pltpu.sync_copy(dst_hbm.at[wid], dst_l)
    plsc.subcore_barrier()

    def step(j, carry):
        pltpu.async_copy(table_hbm.at[src_l.at[j]], rows, sem).wait()
        pltpu.sync_copy(rows, acc.at[dst_l.at[j]], add=True)
        return carry

    lax.fori_loop(0, CH, step, 0)
    plsc.subcore_barrier()
    pltpu.sync_copy(acc.at[pl.ds(sid * RPT, RPT)],
                    out_hbm.at[cid, pl.ds(sid * RPT, RPT)])


def _tc_pre_body(x_ref, w_ref, d0_ref, d1_ref, hwp_ref, dinv_ref):
    deg = d0_ref[...] + d1_ref[...] + 1.0
    dinv = lax.rsqrt(deg)
    hw = jnp.dot(x_ref[...], w_ref[...], preferred_element_type=jnp.float32)
    hwp_ref[...] = hw * dinv
    dinv_ref[...] = dinv


def _tc_layer_body(s0_ref, s1_ref, hwp_ref, dinv_ref, b_ref, w_ref, out_ref):
    dinv = dinv_ref[...]
    s = s0_ref[...].reshape(BR, D) + s1_ref[...].reshape(BR, D)
    h = jnp.maximum(dinv * (s + hwp_ref[...]) + b_ref[...], 0.0)
    out_ref[...] = jnp.dot(h, w_ref[...], preferred_element_type=jnp.float32) * dinv


def _tc_final_body(s0_ref, s1_ref, hwp_ref, dinv_ref, b_ref, wfc_ref, bfc_ref,
                   h_ref, y_ref):
    dinv = dinv_ref[...]
    s = s0_ref[...].reshape(BR, D) + s1_ref[...].reshape(BR, D)
    h = jnp.maximum(dinv * (s + hwp_ref[...]) + b_ref[...], 0.0)
    h_ref[...] = h
    logits = jnp.dot(h, wfc_ref[...], preferred_element_type=jnp.float32) + bfc_ref[...]
    m = jnp.max(logits, axis=-1, keepdims=True)
    e = jnp.exp(logits - m)
    y_ref[...] = e / jnp.sum(e, axis=-1, keepdims=True)


_GRID = (N // BR,)
_row = lambda i: (i, 0)
_fix = lambda i: (0, 0)

_tc_pre = pl.pallas_call(
    _tc_pre_body,
    grid=_GRID,
    in_specs=[
        pl.BlockSpec((BR, D), _row),
        pl.BlockSpec((D, D), _fix),
        pl.BlockSpec((BR, 1), _row),
        pl.BlockSpec((BR, 1), _row),
    ],
    out_specs=[pl.BlockSpec((BR, D), _row), pl.BlockSpec((BR, 1), _row)],
    out_shape=[
        jax.ShapeDtypeStruct((N, D), jnp.float32),
        jax.ShapeDtypeStruct((N, 1), jnp.float32),
    ],
)

_tc_layer = pl.pallas_call(
    _tc_layer_body,
    grid=_GRID,
    in_specs=[
        pl.BlockSpec((1, BR, D), lambda i: (0, i, 0)),
        pl.BlockSpec((1, BR, D), lambda i: (1, i, 0)),
        pl.BlockSpec((BR, D), _row),
        pl.BlockSpec((BR, 1), _row),
        pl.BlockSpec((1, D), _fix),
        pl.BlockSpec((D, D), _fix),
    ],
    out_specs=pl.BlockSpec((BR, D), _row),
    out_shape=jax.ShapeDtypeStruct((N, D), jnp.float32),
)

_tc_final = pl.pallas_call(
    _tc_final_body,
    grid=_GRID,
    in_specs=[
        pl.BlockSpec((1, BR, D), lambda i: (0, i, 0)),
        pl.BlockSpec((1, BR, D), lambda i: (1, i, 0)),
        pl.BlockSpec((BR, D), _row),
        pl.BlockSpec((BR, 1), _row),
        pl.BlockSpec((1, D), _fix),
        pl.BlockSpec((D, C), _fix),
        pl.BlockSpec((1, C), _fix),
    ],
    out_specs=[pl.BlockSpec((BR, D), _row), pl.BlockSpec((BR, C), _row)],
    out_shape=[
        jax.ShapeDtypeStruct((N, D), jnp.float32),
        jax.ShapeDtypeStruct((N, C), jnp.float32),
    ],
)


def kernel(x, edge_index, W1, b1, W2, b2, Wfc, bfc):
    src = edge_index[0].reshape(NW, CH, K)
    dst = edge_index[1].reshape(NW, CH, K)
    degp = _sc_degree(dst)
    d0 = degp[0, :N].reshape(N, 1)
    d1 = degp[1, :N].reshape(N, 1)
    hwp1, dinv = _tc_pre(x, W1, d0, d1)
    s1 = _sc_gather_scatter(hwp1, src, dst)
    hwp2 = _tc_layer(s1, s1, hwp1, dinv, b1.reshape(1, D), W2)
    s2 = _sc_gather_scatter(hwp2, src, dst)
    h, y = _tc_final(s2, s2, hwp2, dinv, b2.reshape(1, D), Wfc, bfc.reshape(1, C))
    return (h, y)


# speedup vs baseline: 32.1091x; 1.6371x over previous
import functools

import jax
import jax.numpy as jnp
from jax import lax
from jax.experimental import pallas as pl
from jax.experimental.pallas import tpu as pltpu
from jax.experimental.pallas import tpu_sc as plsc

N = 10000
D = 128
C = 16
E = 320000
NC, NS = 2, 16
NW = NC * NS
EPT = 10240
EPAD = NW * EPT - E
K = 128
CH = EPT // K
NPASS = 2
PCH = CH // NPASS
NP = 10240
RPT = NP // NS
NDP = 10240
DPT = NDP // NS

BR = 1000

_mesh = plsc.VectorSubcoreMesh(core_axis_name="c", subcore_axis_name="s")


@functools.partial(
    pl.kernel,
    mesh=_mesh,
    out_type=jax.ShapeDtypeStruct((NC, NDP), jnp.float32),
    scratch_types=[
        pltpu.VMEM_SHARED((NDP,), jnp.float32),
        pltpu.VMEM((NPASS, PCH, K), jnp.int32),
        pltpu.VMEM((DPT,), jnp.float32),
        pltpu.VMEM((K,), jnp.float32),
    ],
)
def _sc_degree(dst_hbm, out_hbm, acc, dst_l, zbuf, ones_l):
    cid = lax.axis_index("c")
    sid = lax.axis_index("s")
    wid = cid * NS + sid
    for i in range(DPT // 16):
        zbuf[pl.ds(i * 16, 16)] = jnp.zeros((16,), jnp.float32)
    for i in range(K // 16):
        ones_l[pl.ds(i * 16, 16)] = jnp.ones((16,), jnp.float32)
    pltpu.sync_copy(zbuf, acc.at[pl.ds(sid * DPT, DPT)])
    pltpu.sync_copy(dst_hbm.at[wid], dst_l)
    plsc.subcore_barrier()

    for pa in range(NPASS):
        def step(j, carry):
            pltpu.sync_copy(ones_l, acc.at[dst_l.at[pa, j]], add=True)
            return carry

        lax.fori_loop(0, PCH, step, 0)
    plsc.subcore_barrier()
    pltpu.sync_copy(acc.at[pl.ds(sid * DPT, DPT)],
                    out_hbm.at[cid, pl.ds(sid * DPT, DPT)])


@functools.partial(
    pl.kernel,
    mesh=_mesh,
    out_type=jax.ShapeDtypeStruct((NC, NP, D), jnp.float32),
    scratch_types=[
        pltpu.VMEM_SHARED((NP, D), jnp.float32),
        pltpu.VMEM((PCH, K), jnp.int32),
        pltpu.VMEM((PCH, K), jnp.int32),
        pltpu.VMEM((K, D), jnp.float32),
        pltpu.VMEM((K, D), jnp.float32),
        pltpu.SemaphoreType.DMA,
        pltpu.SemaphoreType.DMA,
    ],
)
def _sc_gather_scatter(table_hbm, src_hbm, dst_hbm, out_hbm,
                       acc, src_l, dst_l, rows_a, rows_b, sem_a, sem_b):
    cid = lax.axis_index("c")
    sid = lax.axis_index("s")
    wid = cid * NS + sid

    def zfill(i, carry):
        for c in range(D // 16):
            rows_a[i, pl.ds(c * 16, 16)] = jnp.zeros((16,), jnp.float32)
        return carry

    lax.fori_loop(0, K, zfill, 0)
    for k in range(RPT // K):
        pltpu.sync_copy(rows_a, acc.at[pl.ds(sid * RPT + k * K, K)])
    plsc.subcore_barrier()

    for pa in range(NPASS):
        pltpu.sync_copy(src_hbm.at[wid, pa], src_l)
        pltpu.sync_copy(dst_hbm.at[wid, pa], dst_l)
        pltpu.async_copy(table_hbm.at[src_l.at[0]], rows_a, sem_a)
        pltpu.async_copy(table_hbm.at[src_l.at[1]], rows_b, sem_b)

        def step(p, carry):
            j = 2 * p
            pltpu.make_async_copy(table_hbm.at[src_l.at[j]], rows_a, sem_a).wait()
            pltpu.sync_copy(rows_a, acc.at[dst_l.at[j]], add=True)
            pltpu.async_copy(table_hbm.at[src_l.at[j + 2]], rows_a, sem_a)
            pltpu.make_async_copy(table_hbm.at[src_l.at[j + 1]], rows_b, sem_b).wait()
            pltpu.sync_copy(rows_b, acc.at[dst_l.at[j + 1]], add=True)
            pltpu.async_copy(table_hbm.at[src_l.at[j + 3]], rows_b, sem_b)
            return carry

        lax.fori_loop(0, PCH // 2 - 1, step, 0)
        pltpu.make_async_copy(table_hbm.at[src_l.at[PCH - 2]], rows_a, sem_a).wait()
        pltpu.sync_copy(rows_a, acc.at[dst_l.at[PCH - 2]], add=True)
        pltpu.make_async_copy(table_hbm.at[src_l.at[PCH - 1]], rows_b, sem_b).wait()
        pltpu.sync_copy(rows_b, acc.at[dst_l.at[PCH - 1]], add=True)
    plsc.subcore_barrier()
    pltpu.sync_copy(acc.at[pl.ds(sid * RPT, RPT)],
                    out_hbm.at[cid, pl.ds(sid * RPT, RPT)])


def _tc_pre_body(x_ref, w_ref, d0_ref, d1_ref, hwp_ref, dinv_ref):
    deg = d0_ref[...] + d1_ref[...] + 1.0
    dinv = lax.rsqrt(deg)
    hw = jnp.dot(x_ref[...], w_ref[...], preferred_element_type=jnp.float32)
    hwp_ref[...] = hw * dinv
    dinv_ref[...] = dinv


def _tc_layer_body(s0_ref, s1_ref, hwp_ref, dinv_ref, b_ref, w_ref, out_ref):
    dinv = dinv_ref[...]
    s = s0_ref[...].reshape(BR, D) + s1_ref[...].reshape(BR, D)
    h = jnp.maximum(dinv * (s + hwp_ref[...]) + b_ref[...], 0.0)
    out_ref[...] = jnp.dot(h, w_ref[...], preferred_element_type=jnp.float32) * dinv


def _tc_final_body(s0_ref, s1_ref, hwp_ref, dinv_ref, b_ref, wfc_ref, bfc_ref,
                   h_ref, y_ref):
    dinv = dinv_ref[...]
    s = s0_ref[...].reshape(BR, D) + s1_ref[...].reshape(BR, D)
    h = jnp.maximum(dinv * (s + hwp_ref[...]) + b_ref[...], 0.0)
    h_ref[...] = h
    logits = jnp.dot(h, wfc_ref[...], preferred_element_type=jnp.float32) + bfc_ref[...]
    m = jnp.max(logits, axis=-1, keepdims=True)
    e = jnp.exp(logits - m)
    y_ref[...] = e / jnp.sum(e, axis=-1, keepdims=True)


_GRID = (N // BR,)
_row = lambda i: (i, 0)
_fix = lambda i: (0, 0)

_tc_pre = pl.pallas_call(
    _tc_pre_body,
    grid=_GRID,
    in_specs=[
        pl.BlockSpec((BR, D), _row),
        pl.BlockSpec((D, D), _fix),
        pl.BlockSpec((BR, 1), _row),
        pl.BlockSpec((BR, 1), _row),
    ],
    out_specs=[pl.BlockSpec((BR, D), _row), pl.BlockSpec((BR, 1), _row)],
    out_shape=[
        jax.ShapeDtypeStruct((N, D), jnp.float32),
        jax.ShapeDtypeStruct((N, 1), jnp.float32),
    ],
)

_tc_layer = pl.pallas_call(
    _tc_layer_body,
    grid=_GRID,
    in_specs=[
        pl.BlockSpec((1, BR, D), lambda i: (0, i, 0)),
        pl.BlockSpec((1, BR, D), lambda i: (1, i, 0)),
        pl.BlockSpec((BR, D), _row),
        pl.BlockSpec((BR, 1), _row),
        pl.BlockSpec((1, D), _fix),
        pl.BlockSpec((D, D), _fix),
    ],
    out_specs=pl.BlockSpec((BR, D), _row),
    out_shape=jax.ShapeDtypeStruct((N, D), jnp.float32),
)

_tc_final = pl.pallas_call(
    _tc_final_body,
    grid=_GRID,
    in_specs=[
        pl.BlockSpec((1, BR, D), lambda i: (0, i, 0)),
        pl.BlockSpec((1, BR, D), lambda i: (1, i, 0)),
        pl.BlockSpec((BR, D), _row),
        pl.BlockSpec((BR, 1), _row),
        pl.BlockSpec((1, D), _fix),
        pl.BlockSpec((D, C), _fix),
        pl.BlockSpec((1, C), _fix),
    ],
    out_specs=[pl.BlockSpec((BR, D), _row), pl.BlockSpec((BR, C), _row)],
    out_shape=[
        jax.ShapeDtypeStruct((N, D), jnp.float32),
        jax.ShapeDtypeStruct((N, C), jnp.float32),
    ],
)


def kernel(x, edge_index, W1, b1, W2, b2, Wfc, bfc):
    pad = jnp.arange(EPAD, dtype=jnp.int32) % (NP - N)
    src = jnp.concatenate([edge_index[0], pad]).reshape(NW, NPASS, PCH, K)
    dst = jnp.concatenate([edge_index[1], N + pad]).reshape(NW, NPASS, PCH, K)
    degp = _sc_degree(dst)
    d0 = degp[0].reshape(NDP, 1)
    d1 = degp[1].reshape(NDP, 1)
    hwp1, dinv = _tc_pre(x, W1, d0, d1)
    s1 = _sc_gather_scatter(hwp1, src, dst)
    hwp2 = _tc_layer(s1, s1, hwp1, dinv, b1.reshape(1, D), W2)
    s2 = _sc_gather_scatter(hwp2, src, dst)
    h, y = _tc_final(s2, s2, hwp2, dinv, b2.reshape(1, D), Wfc, bfc.reshape(1, C))
    return (h, y)


# speedup vs baseline: 32.7557x; 1.0201x over previous
import functools

import jax
import jax.numpy as jnp
from jax import lax
from jax.experimental import pallas as pl
from jax.experimental.pallas import tpu as pltpu
from jax.experimental.pallas import tpu_sc as plsc

N = 10000
D = 128
C = 16
E = 320000
NC, NS = 2, 16
NW = NC * NS
EPT = 10240
EPAD = NW * EPT - E
K = 128
CH = EPT // K
NPASS = 2
PCH = CH // NPASS
NP = 10240
RPT = NP // NS
NDP = 10240
DPT = NDP // NS

BR = 2000

_mesh = plsc.VectorSubcoreMesh(core_axis_name="c", subcore_axis_name="s")


@functools.partial(
    pl.kernel,
    mesh=_mesh,
    out_type=jax.ShapeDtypeStruct((NC, NDP), jnp.float32),
    scratch_types=[
        pltpu.VMEM_SHARED((NDP,), jnp.float32),
        pltpu.VMEM((NPASS, PCH, K), jnp.int32),
        pltpu.VMEM((DPT,), jnp.float32),
        pltpu.VMEM((K,), jnp.float32),
    ],
)
def _sc_degree(dst_hbm, out_hbm, acc, dst_l, zbuf, ones_l):
    cid = lax.axis_index("c")
    sid = lax.axis_index("s")
    wid = cid * NS + sid
    for i in range(DPT // 16):
        zbuf[pl.ds(i * 16, 16)] = jnp.zeros((16,), jnp.float32)
    for i in range(K // 16):
        ones_l[pl.ds(i * 16, 16)] = jnp.ones((16,), jnp.float32)
    pltpu.sync_copy(zbuf, acc.at[pl.ds(sid * DPT, DPT)])
    pltpu.sync_copy(dst_hbm.at[wid], dst_l)
    plsc.subcore_barrier()

    for pa in range(NPASS):
        def step(j, carry):
            pltpu.sync_copy(ones_l, acc.at[dst_l.at[pa, j]], add=True)
            return carry

        lax.fori_loop(0, PCH, step, 0)
    plsc.subcore_barrier()
    pltpu.sync_copy(acc.at[pl.ds(sid * DPT, DPT)],
                    out_hbm.at[cid, pl.ds(sid * DPT, DPT)])


@functools.partial(
    pl.kernel,
    mesh=_mesh,
    out_type=jax.ShapeDtypeStruct((NC, NP, D), jnp.float32),
    scratch_types=[
        pltpu.VMEM_SHARED((NP, D), jnp.float32),
        pltpu.VMEM((PCH, K), jnp.int32),
        pltpu.VMEM((PCH, K), jnp.int32),
        pltpu.VMEM((K, D), jnp.float32),
        pltpu.VMEM((K, D), jnp.float32),
        pltpu.SemaphoreType.DMA,
        pltpu.SemaphoreType.DMA,
    ],
)
def _sc_gather_scatter(table_hbm, src_hbm, dst_hbm, out_hbm,
                       acc, src_l, dst_l, rows_a, rows_b, sem_a, sem_b):
    cid = lax.axis_index("c")
    sid = lax.axis_index("s")
    wid = cid * NS + sid

    def zfill(i, carry):
        for c in range(D // 16):
            rows_a[i, pl.ds(c * 16, 16)] = jnp.zeros((16,), jnp.float32)
        return carry

    lax.fori_loop(0, K, zfill, 0)
    for k in range(RPT // K):
        pltpu.sync_copy(rows_a, acc.at[pl.ds(sid * RPT + k * K, K)])
    pltpu.sync_copy(src_hbm.at[wid, 0], src_l)
    pltpu.sync_copy(dst_hbm.at[wid, 0], dst_l)
    pltpu.async_copy(table_hbm.at[src_l.at[0]], rows_a, sem_a)
    pltpu.async_copy(table_hbm.at[src_l.at[1]], rows_b, sem_b)
    plsc.subcore_barrier()

    for pa in range(NPASS):
        if pa > 0:
            pltpu.sync_copy(src_hbm.at[wid, pa], src_l)
            pltpu.sync_copy(dst_hbm.at[wid, pa], dst_l)
            pltpu.async_copy(table_hbm.at[src_l.at[0]], rows_a, sem_a)
            pltpu.async_copy(table_hbm.at[src_l.at[1]], rows_b, sem_b)

        def step(p, carry):
            j = 2 * p
            pltpu.make_async_copy(table_hbm.at[src_l.at[j]], rows_a, sem_a).wait()
            pltpu.sync_copy(rows_a, acc.at[dst_l.at[j]], add=True)
            pltpu.async_copy(table_hbm.at[src_l.at[j + 2]], rows_a, sem_a)
            pltpu.make_async_copy(table_hbm.at[src_l.at[j + 1]], rows_b, sem_b).wait()
            pltpu.sync_copy(rows_b, acc.at[dst_l.at[j + 1]], add=True)
            pltpu.async_copy(table_hbm.at[src_l.at[j + 3]], rows_b, sem_b)
            return carry

        lax.fori_loop(0, PCH // 2 - 1, step, 0)
        pltpu.make_async_copy(table_hbm.at[src_l.at[PCH - 2]], rows_a, sem_a).wait()
        pltpu.sync_copy(rows_a, acc.at[dst_l.at[PCH - 2]], add=True)
        pltpu.make_async_copy(table_hbm.at[src_l.at[PCH - 1]], rows_b, sem_b).wait()
        pltpu.sync_copy(rows_b, acc.at[dst_l.at[PCH - 1]], add=True)
    plsc.subcore_barrier()
    pltpu.sync_copy(acc.at[pl.ds(sid * RPT, RPT)],
                    out_hbm.at[cid, pl.ds(sid * RPT, RPT)])


def _tc_pre_body(x_ref, w_ref, d0_ref, d1_ref, hwp_ref, dinv_ref):
    deg = d0_ref[...] + d1_ref[...] + 1.0
    dinv = lax.rsqrt(deg)
    hw = jnp.dot(x_ref[...], w_ref[...], preferred_element_type=jnp.float32)
    hwp_ref[...] = hw * dinv
    dinv_ref[...] = dinv


def _tc_layer_body(s0_ref, s1_ref, hwp_ref, dinv_ref, b_ref, w_ref, out_ref):
    dinv = dinv_ref[...]
    s = s0_ref[...].reshape(BR, D) + s1_ref[...].reshape(BR, D)
    h = jnp.maximum(dinv * (s + hwp_ref[...]) + b_ref[...], 0.0)
    out_ref[...] = jnp.dot(h, w_ref[...], preferred_element_type=jnp.float32) * dinv


def _tc_final_body(s0_ref, s1_ref, hwp_ref, dinv_ref, b_ref, wfc_ref, bfc_ref,
                   h_ref, y_ref):
    dinv = dinv_ref[...]
    s = s0_ref[...].reshape(BR, D) + s1_ref[...].reshape(BR, D)
    h = jnp.maximum(dinv * (s + hwp_ref[...]) + b_ref[...], 0.0)
    h_ref[...] = h
    logits = jnp.dot(h, wfc_ref[...], preferred_element_type=jnp.float32) + bfc_ref[...]
    m = jnp.max(logits, axis=-1, keepdims=True)
    e = jnp.exp(logits - m)
    y_ref[...] = e / jnp.sum(e, axis=-1, keepdims=True)


_GRID = (N // BR,)
_row = lambda i: (i, 0)
_fix = lambda i: (0, 0)

_tc_pre = pl.pallas_call(
    _tc_pre_body,
    grid=_GRID,
    in_specs=[
        pl.BlockSpec((BR, D), _row),
        pl.BlockSpec((D, D), _fix),
        pl.BlockSpec((BR, 1), _row),
        pl.BlockSpec((BR, 1), _row),
    ],
    out_specs=[pl.BlockSpec((BR, D), _row), pl.BlockSpec((BR, 1), _row)],
    out_shape=[
        jax.ShapeDtypeStruct((N, D), jnp.float32),
        jax.ShapeDtypeStruct((N, 1), jnp.float32),
    ],
)

_tc_layer = pl.pallas_call(
    _tc_layer_body,
    grid=_GRID,
    in_specs=[
        pl.BlockSpec((1, BR, D), lambda i: (0, i, 0)),
        pl.BlockSpec((1, BR, D), lambda i: (1, i, 0)),
        pl.BlockSpec((BR, D), _row),
        pl.BlockSpec((BR, 1), _row),
        pl.BlockSpec((1, D), _fix),
        pl.BlockSpec((D, D), _fix),
    ],
    out_specs=pl.BlockSpec((BR, D), _row),
    out_shape=jax.ShapeDtypeStruct((N, D), jnp.float32),
)

_tc_final = pl.pallas_call(
    _tc_final_body,
    grid=_GRID,
    in_specs=[
        pl.BlockSpec((1, BR, D), lambda i: (0, i, 0)),
        pl.BlockSpec((1, BR, D), lambda i: (1, i, 0)),
        pl.BlockSpec((BR, D), _row),
        pl.BlockSpec((BR, 1), _row),
        pl.BlockSpec((1, D), _fix),
        pl.BlockSpec((D, C), _fix),
        pl.BlockSpec((1, C), _fix),
    ],
    out_specs=[pl.BlockSpec((BR, D), _row), pl.BlockSpec((BR, C), _row)],
    out_shape=[
        jax.ShapeDtypeStruct((N, D), jnp.float32),
        jax.ShapeDtypeStruct((N, C), jnp.float32),
    ],
)


def kernel(x, edge_index, W1, b1, W2, b2, Wfc, bfc):
    pad = jnp.arange(EPAD, dtype=jnp.int32) % (NP - N)
    src = jnp.concatenate([edge_index[0], pad]).reshape(NW, NPASS, PCH, K)
    dst = jnp.concatenate([edge_index[1], N + pad]).reshape(NW, NPASS, PCH, K)
    degp = _sc_degree(dst)
    d0 = degp[0].reshape(NDP, 1)
    d1 = degp[1].reshape(NDP, 1)
    hwp1, dinv = _tc_pre(x, W1, d0, d1)
    s1 = _sc_gather_scatter(hwp1, src, dst)
    hwp2 = _tc_layer(s1, s1, hwp1, dinv, b1.reshape(1, D), W2)
    s2 = _sc_gather_scatter(hwp2, src, dst)
    h, y = _tc_final(s2, s2, hwp2, dinv, b2.reshape(1, D), Wfc, bfc.reshape(1, C))
    return (h, y)


# speedup vs baseline: 33.0669x; 1.0095x over previous
import functools

import jax
import jax.numpy as jnp
from jax import lax
from jax.experimental import pallas as pl
from jax.experimental.pallas import tpu as pltpu
from jax.experimental.pallas import tpu_sc as plsc

N = 10000
D = 128
C = 16
E = 320000
NC, NS = 2, 16
NW = NC * NS
EPT = 10240
EPAD = NW * EPT - E
K = 128
CH = EPT // K
NPASS = 2
PCH = CH // NPASS
NP = 10240
RPT = NP // NS
NDP = 10240
DPT = NDP // NS

BR = 5000

_mesh = plsc.VectorSubcoreMesh(core_axis_name="c", subcore_axis_name="s")


@functools.partial(
    pl.kernel,
    mesh=_mesh,
    out_type=jax.ShapeDtypeStruct((NC, NDP), jnp.float32),
    scratch_types=[
        pltpu.VMEM_SHARED((NDP,), jnp.float32),
        pltpu.VMEM((NPASS, PCH, K), jnp.int32),
        pltpu.VMEM((DPT,), jnp.float32),
        pltpu.VMEM((K,), jnp.float32),
    ],
)
def _sc_degree(dst_hbm, out_hbm, acc, dst_l, zbuf, ones_l):
    cid = lax.axis_index("c")
    sid = lax.axis_index("s")
    wid = cid * NS + sid
    for i in range(DPT // 16):
        zbuf[pl.ds(i * 16, 16)] = jnp.zeros((16,), jnp.float32)
    for i in range(K // 16):
        ones_l[pl.ds(i * 16, 16)] = jnp.ones((16,), jnp.float32)
    pltpu.sync_copy(zbuf, acc.at[pl.ds(sid * DPT, DPT)])
    pltpu.sync_copy(dst_hbm.at[wid], dst_l)
    plsc.subcore_barrier()

    for pa in range(NPASS):
        def step(j, carry):
            pltpu.sync_copy(ones_l, acc.at[dst_l.at[pa, j]], add=True)
            return carry

        lax.fori_loop(0, PCH, step, 0)
    plsc.subcore_barrier()
    pltpu.sync_copy(acc.at[pl.ds(sid * DPT, DPT)],
                    out_hbm.at[cid, pl.ds(sid * DPT, DPT)])


@functools.partial(
    pl.kernel,
    mesh=_mesh,
    out_type=jax.ShapeDtypeStruct((NC, NP, D), jnp.float32),
    scratch_types=[
        pltpu.VMEM_SHARED((NP, D), jnp.float32),
        pltpu.VMEM((PCH, K), jnp.int32),
        pltpu.VMEM((PCH, K), jnp.int32),
        pltpu.VMEM((K, D), jnp.float32),
        pltpu.VMEM((K, D), jnp.float32),
        pltpu.SemaphoreType.DMA,
        pltpu.SemaphoreType.DMA,
    ],
)
def _sc_gather_scatter(table_hbm, src_hbm, dst_hbm, out_hbm,
                       acc, src_l, dst_l, rows_a, rows_b, sem_a, sem_b):
    cid = lax.axis_index("c")
    sid = lax.axis_index("s")
    wid = cid * NS + sid

    def zfill(i, carry):
        for c in range(D // 16):
            rows_a[i, pl.ds(c * 16, 16)] = jnp.zeros((16,), jnp.float32)
        return carry

    lax.fori_loop(0, K, zfill, 0)
    for k in range(RPT // K):
        pltpu.sync_copy(rows_a, acc.at[pl.ds(sid * RPT + k * K, K)])
    pltpu.sync_copy(src_hbm.at[wid, 0], src_l)
    pltpu.sync_copy(dst_hbm.at[wid, 0], dst_l)
    pltpu.async_copy(table_hbm.at[src_l.at[0]], rows_a, sem_a)
    pltpu.async_copy(table_hbm.at[src_l.at[1]], rows_b, sem_b)
    plsc.subcore_barrier()

    for pa in range(NPASS):
        if pa > 0:
            pltpu.sync_copy(src_hbm.at[wid, pa], src_l)
            pltpu.sync_copy(dst_hbm.at[wid, pa], dst_l)
            pltpu.async_copy(table_hbm.at[src_l.at[0]], rows_a, sem_a)
            pltpu.async_copy(table_hbm.at[src_l.at[1]], rows_b, sem_b)

        def step(p, carry):
            j = 2 * p
            pltpu.make_async_copy(table_hbm.at[src_l.at[j]], rows_a, sem_a).wait()
            pltpu.sync_copy(rows_a, acc.at[dst_l.at[j]], add=True)
            pltpu.async_copy(table_hbm.at[src_l.at[j + 2]], rows_a, sem_a)
            pltpu.make_async_copy(table_hbm.at[src_l.at[j + 1]], rows_b, sem_b).wait()
            pltpu.sync_copy(rows_b, acc.at[dst_l.at[j + 1]], add=True)
            pltpu.async_copy(table_hbm.at[src_l.at[j + 3]], rows_b, sem_b)
            return carry

        lax.fori_loop(0, PCH // 2 - 1, step, 0)
        pltpu.make_async_copy(table_hbm.at[src_l.at[PCH - 2]], rows_a, sem_a).wait()
        pltpu.sync_copy(rows_a, acc.at[dst_l.at[PCH - 2]], add=True)
        pltpu.make_async_copy(table_hbm.at[src_l.at[PCH - 1]], rows_b, sem_b).wait()
        pltpu.sync_copy(rows_b, acc.at[dst_l.at[PCH - 1]], add=True)
    plsc.subcore_barrier()
    pltpu.sync_copy(acc.at[pl.ds(sid * RPT, RPT)],
                    out_hbm.at[cid, pl.ds(sid * RPT, RPT)])


def _tc_pre_body(x_ref, w_ref, d0_ref, d1_ref, hwp_ref, dinv_ref):
    deg = d0_ref[...] + d1_ref[...] + 1.0
    dinv = lax.rsqrt(deg)
    hw = jnp.dot(x_ref[...], w_ref[...], preferred_element_type=jnp.float32)
    hwp_ref[...] = hw * dinv
    dinv_ref[...] = dinv


def _tc_layer_body(s0_ref, s1_ref, hwp_ref, dinv_ref, b_ref, w_ref, out_ref):
    dinv = dinv_ref[...]
    s = s0_ref[...].reshape(BR, D) + s1_ref[...].reshape(BR, D)
    h = jnp.maximum(dinv * (s + hwp_ref[...]) + b_ref[...], 0.0)
    out_ref[...] = jnp.dot(h, w_ref[...], preferred_element_type=jnp.float32) * dinv


def _tc_final_body(s0_ref, s1_ref, hwp_ref, dinv_ref, b_ref, wfc_ref, bfc_ref,
                   h_ref, y_ref):
    dinv = dinv_ref[...]
    s = s0_ref[...].reshape(BR, D) + s1_ref[...].reshape(BR, D)
    h = jnp.maximum(dinv * (s + hwp_ref[...]) + b_ref[...], 0.0)
    h_ref[...] = h
    logits = jnp.dot(h, wfc_ref[...], preferred_element_type=jnp.float32) + bfc_ref[...]
    m = jnp.max(logits, axis=-1, keepdims=True)
    e = jnp.exp(logits - m)
    y_ref[...] = e / jnp.sum(e, axis=-1, keepdims=True)


_GRID = (N // BR,)
_row = lambda i: (i, 0)
_fix = lambda i: (0, 0)

_tc_pre = pl.pallas_call(
    _tc_pre_body,
    grid=_GRID,
    in_specs=[
        pl.BlockSpec((BR, D), _row),
        pl.BlockSpec((D, D), _fix),
        pl.BlockSpec((BR, 1), _row),
        pl.BlockSpec((BR, 1), _row),
    ],
    out_specs=[pl.BlockSpec((BR, D), _row), pl.BlockSpec((BR, 1), _row)],
    out_shape=[
        jax.ShapeDtypeStruct((N, D), jnp.float32),
        jax.ShapeDtypeStruct((N, 1), jnp.float32),
    ],
)

_tc_layer = pl.pallas_call(
    _tc_layer_body,
    grid=_GRID,
    in_specs=[
        pl.BlockSpec((1, BR, D), lambda i: (0, i, 0)),
        pl.BlockSpec((1, BR, D), lambda i: (1, i, 0)),
        pl.BlockSpec((BR, D), _row),
        pl.BlockSpec((BR, 1), _row),
        pl.BlockSpec((1, D), _fix),
        pl.BlockSpec((D, D), _fix),
    ],
    out_specs=pl.BlockSpec((BR, D), _row),
    out_shape=jax.ShapeDtypeStruct((N, D), jnp.float32),
)

_tc_final = pl.pallas_call(
    _tc_final_body,
    grid=_GRID,
    in_specs=[
        pl.BlockSpec((1, BR, D), lambda i: (0, i, 0)),
        pl.BlockSpec((1, BR, D), lambda i: (1, i, 0)),
        pl.BlockSpec((BR, D), _row),
        pl.BlockSpec((BR, 1), _row),
        pl.BlockSpec((1, D), _fix),
        pl.BlockSpec((D, C), _fix),
        pl.BlockSpec((1, C), _fix),
    ],
    out_specs=[pl.BlockSpec((BR, D), _row), pl.BlockSpec((BR, C), _row)],
    out_shape=[
        jax.ShapeDtypeStruct((N, D), jnp.float32),
        jax.ShapeDtypeStruct((N, C), jnp.float32),
    ],
)


def kernel(x, edge_index, W1, b1, W2, b2, Wfc, bfc):
    pad = jnp.arange(EPAD, dtype=jnp.int32) % (NP - N)
    src = jnp.concatenate([edge_index[0], pad]).reshape(NW, NPASS, PCH, K)
    dst = jnp.concatenate([edge_index[1], N + pad]).reshape(NW, NPASS, PCH, K)
    degp = _sc_degree(dst)
    d0 = degp[0].reshape(NDP, 1)
    d1 = degp[1].reshape(NDP, 1)
    hwp1, dinv = _tc_pre(x, W1, d0, d1)
    s1 = _sc_gather_scatter(hwp1, src, dst)
    hwp2 = _tc_layer(s1, s1, hwp1, dinv, b1.reshape(1, D), W2)
    s2 = _sc_gather_scatter(hwp2, src, dst)
    h, y = _tc_final(s2, s2, hwp2, dinv, b2.reshape(1, D), Wfc, bfc.reshape(1, C))
    return (h, y)
